# baseline (speedup 1.0000x reference)
import functools

import jax
import jax.numpy as jnp
from jax import lax
from jax.experimental import pallas as pl
from jax.experimental.pallas import tpu as pltpu
from jax.experimental.pallas import tpu_sc as plsc

NIMG = 16
ROWS = 512
COLS = 512
K_SC = 4
K_TC = NIMG - K_SC
NC = 2
NS = 16
L = 16
NW = NC * NS
ROWS_W = K_SC * ROWS // NW
CHUNKR = 32
NCHUNK = ROWS_W // CHUNKR
BANKS = 4
STEP = BANKS * L

_mesh = plsc.VectorSubcoreMesh(core_axis_name="c", subcore_axis_name="s")


@functools.partial(
    pl.kernel,
    mesh=_mesh,
    out_type=jax.ShapeDtypeStruct((NW, 4, L), jnp.float32),
    scratch_types=[
        pltpu.VMEM((2, CHUNKR, COLS), jnp.float32),
        pltpu.VMEM((2, CHUNKR, COLS), jnp.float32),
        pltpu.VMEM((4, L), jnp.float32),
        pltpu.SemaphoreType.DMA,
        pltpu.SemaphoreType.DMA,
        pltpu.SemaphoreType.DMA,
        pltpu.SemaphoreType.DMA,
    ],
)
def _flood_mse_sc(a_hbm, b_hbm, out_hbm, a_v, b_v, res_v,
                  sa0, sa1, sb0, sb1):
    wid = lax.axis_index("s") * NC + lax.axis_index("c")
    grow0 = wid * ROWS_W
    sems_a = (sa0, sa1)
    sems_b = (sb0, sb1)

    def dma_pair(c, slot):
        gr = grow0 + c * CHUNKR
        img = gr // ROWS
        r = gr % ROWS
        ca = pltpu.make_async_copy(
            a_hbm.at[img, 0, pl.ds(r, CHUNKR), :], a_v.at[slot],
            sems_a[slot])
        cb = pltpu.make_async_copy(
            b_hbm.at[img, 0, pl.ds(r, CHUNKR), :], b_v.at[slot],
            sems_b[slot])
        return ca, cb

    for c in (0, 1):
        ca, cb = dma_pair(c, c)
        ca.start()
        cb.start()

    zf = jnp.zeros((L,), jnp.float32)
    zi = jnp.zeros((L,), jnp.int32)
    accs = tuple((zf, zi, zf, zi) for _ in range(BANKS))

    for c in range(NCHUNK):
        slot = c % 2
        ca, cb = dma_pair(c, slot)
        ca.wait()
        cb.wait()
        av = a_v.at[slot]
        bv = b_v.at[slot]

        @plsc.parallel_loop(0, CHUNKR * COLS, STEP, unroll=1, carry=accs)
        def accs(i, banks):
            r = jax.lax.shift_right_logical(i, 9)
            col = pl.multiple_of(jax.lax.bitwise_and(i, COLS - 1), STEP)
            out = []
            for j in range(BANKS):
                sl, cl, sp, cp = banks[j]
                a = av[r, pl.ds(col + j * L, L)]
                b = bv[r, pl.ds(col + j * L, L)]
                d = a - b
                sq = d * d
                ml = b > 0.0
                mp = a > 0.0
                sl = sl + jnp.where(ml, sq, 0.0)
                sp = sp + jnp.where(mp, sq, 0.0)
                cl = cl + jnp.where(ml, 1, 0)
                cp = cp + jnp.where(mp, 1, 0)
                out.append((sl, cl, sp, cp))
            return tuple(out)

        if c + 2 < NCHUNK:
            na, nb = dma_pair(c + 2, slot)
            na.start()
            nb.start()

    sl, cl, sp, cp = accs[0]
    for j in range(1, BANKS):
        sl = sl + accs[j][0]
        cl = cl + accs[j][1]
        sp = sp + accs[j][2]
        cp = cp + accs[j][3]
    res_v[0, :] = sl
    res_v[1, :] = cl.astype(jnp.float32)
    res_v[2, :] = sp
    res_v[3, :] = cp.astype(jnp.float32)
    pltpu.sync_copy(res_v, out_hbm.at[wid])


IMG_B = 4


def _flood_mse_tc_body(a_ref, b_ref, o_ref):
    i = pl.program_id(0)
    a = a_ref[...].reshape(IMG_B * ROWS, COLS)
    b = b_ref[...].reshape(IMG_B * ROWS, COLS)
    d = a - b
    sq = d * d
    mlf = (b > 0.0).astype(jnp.float32)
    mpf = (a > 0.0).astype(jnp.float32)
    ones = jnp.ones((1, IMG_B * ROWS), jnp.float32)

    def colsum(x):
        return jax.lax.dot_general(
            ones, x, (((1,), (0,)), ((), ())),
            preferred_element_type=jnp.float32)

    sl = jnp.sum(colsum(sq * mlf))
    cl = jnp.sum(colsum(mlf))
    sp = jnp.sum(colsum(sq * mpf))
    cp = jnp.sum(colsum(mpf))

    @pl.when(i == 0)
    def _():
        o_ref[0] = 0.0
        o_ref[1] = 0.0
        o_ref[2] = 0.0
        o_ref[3] = 0.0

    o_ref[0] += sl
    o_ref[1] += cl
    o_ref[2] += sp
    o_ref[3] += cp


_flood_mse_tc = pl.pallas_call(
    _flood_mse_tc_body,
    grid=(K_TC // IMG_B,),
    in_specs=[
        pl.BlockSpec((IMG_B, 1, ROWS, COLS),
                     lambda i: (K_SC // IMG_B + i, 0, 0, 0)),
        pl.BlockSpec((IMG_B, 1, ROWS, COLS),
                     lambda i: (K_SC // IMG_B + i, 0, 0, 0)),
    ],
    out_specs=pl.BlockSpec(memory_space=pltpu.SMEM),
    out_shape=jax.ShapeDtypeStruct((4,), jnp.float32),
)


def kernel(inputs, targets):
    parts_tc = _flood_mse_tc(inputs, targets)
    parts_sc = _flood_mse_sc(inputs, targets)
    sums = jnp.sum(parts_sc, axis=(0, 2)) + parts_tc
    loss_label = sums[0] / sums[1]
    loss_pred = sums[2] / sums[3]
    return (loss_label + loss_pred, loss_label, loss_pred)


# speedup vs baseline: 1.0124x; 1.0124x over previous
import functools

import jax
import jax.numpy as jnp
from jax import lax
from jax.experimental import pallas as pl
from jax.experimental.pallas import tpu as pltpu
from jax.experimental.pallas import tpu_sc as plsc

NIMG = 16
ROWS = 512
COLS = 512
K_SC = 4
K_TC = NIMG - K_SC
NC = 2
NS = 16
L = 16
NW = NC * NS
ROWS_W = K_SC * ROWS // NW
CHUNKR = 32
NCHUNK = ROWS_W // CHUNKR
BANKS = 4
STEP = BANKS * L

_mesh = plsc.VectorSubcoreMesh(core_axis_name="c", subcore_axis_name="s")


@functools.partial(
    pl.kernel,
    mesh=_mesh,
    out_type=jax.ShapeDtypeStruct((NW, 4, L), jnp.float32),
    scratch_types=[
        pltpu.VMEM((2, CHUNKR, COLS), jnp.float32),
        pltpu.VMEM((2, CHUNKR, COLS), jnp.float32),
        pltpu.VMEM((4, L), jnp.float32),
        pltpu.SemaphoreType.DMA,
        pltpu.SemaphoreType.DMA,
        pltpu.SemaphoreType.DMA,
        pltpu.SemaphoreType.DMA,
    ],
)
def _flood_mse_sc(a_hbm, b_hbm, out_hbm, a_v, b_v, res_v,
                  sa0, sa1, sb0, sb1):
    wid = lax.axis_index("s") * NC + lax.axis_index("c")
    grow0 = wid * ROWS_W
    sems_a = (sa0, sa1)
    sems_b = (sb0, sb1)

    def dma_pair(c, slot):
        gr = grow0 + c * CHUNKR
        img = gr // ROWS
        r = gr % ROWS
        ca = pltpu.make_async_copy(
            a_hbm.at[img, 0, pl.ds(r, CHUNKR), :], a_v.at[slot],
            sems_a[slot])
        cb = pltpu.make_async_copy(
            b_hbm.at[img, 0, pl.ds(r, CHUNKR), :], b_v.at[slot],
            sems_b[slot])
        return ca, cb

    for c in (0, 1):
        ca, cb = dma_pair(c, c)
        ca.start()
        cb.start()

    zf = jnp.zeros((L,), jnp.float32)
    zi = jnp.zeros((L,), jnp.int32)
    accs = tuple((zf, zi, zf, zi) for _ in range(BANKS))

    for c in range(NCHUNK):
        slot = c % 2
        ca, cb = dma_pair(c, slot)
        ca.wait()
        cb.wait()
        av = a_v.at[slot]
        bv = b_v.at[slot]

        @plsc.parallel_loop(0, CHUNKR * COLS, STEP, unroll=1, carry=accs)
        def accs(i, banks):
            r = jax.lax.shift_right_logical(i, 9)
            col = pl.multiple_of(jax.lax.bitwise_and(i, COLS - 1), STEP)
            out = []
            for j in range(BANKS):
                sl, cl, sp, cp = banks[j]
                a = av[r, pl.ds(col + j * L, L)]
                b = bv[r, pl.ds(col + j * L, L)]
                d = a - b
                sq = d * d
                ml = b > 0.0
                mp = a > 0.0
                sl = sl + jnp.where(ml, sq, 0.0)
                sp = sp + jnp.where(mp, sq, 0.0)
                cl = cl + jnp.where(ml, 1, 0)
                cp = cp + jnp.where(mp, 1, 0)
                out.append((sl, cl, sp, cp))
            return tuple(out)

        if c + 2 < NCHUNK:
            na, nb = dma_pair(c + 2, slot)
            na.start()
            nb.start()

    sl, cl, sp, cp = accs[0]
    for j in range(1, BANKS):
        sl = sl + accs[j][0]
        cl = cl + accs[j][1]
        sp = sp + accs[j][2]
        cp = cp + accs[j][3]
    res_v[0, :] = sl
    res_v[1, :] = cl.astype(jnp.float32)
    res_v[2, :] = sp
    res_v[3, :] = cp.astype(jnp.float32)
    pltpu.sync_copy(res_v, out_hbm.at[wid])


IMG_B = 2


def _flood_mse_tc_body(a_ref, b_ref, o_ref):
    i = pl.program_id(0)
    a = a_ref[...].reshape(IMG_B * ROWS, COLS)
    b = b_ref[...].reshape(IMG_B * ROWS, COLS)
    d = a - b
    sq = d * d
    mlf = (b > 0.0).astype(jnp.float32)
    mpf = (a > 0.0).astype(jnp.float32)
    ones = jnp.ones((1, IMG_B * ROWS), jnp.float32)

    def colsum(x):
        return jax.lax.dot_general(
            ones, x, (((1,), (0,)), ((), ())),
            preferred_element_type=jnp.float32)

    sl = jnp.sum(colsum(sq * mlf))
    cl = jnp.sum(colsum(mlf))
    sp = jnp.sum(colsum(sq * mpf))
    cp = jnp.sum(colsum(mpf))

    @pl.when(i == 0)
    def _():
        o_ref[0] = 0.0
        o_ref[1] = 0.0
        o_ref[2] = 0.0
        o_ref[3] = 0.0

    o_ref[0] += sl
    o_ref[1] += cl
    o_ref[2] += sp
    o_ref[3] += cp


_flood_mse_tc = pl.pallas_call(
    _flood_mse_tc_body,
    grid=(K_TC // IMG_B,),
    in_specs=[
        pl.BlockSpec((IMG_B, 1, ROWS, COLS),
                     lambda i: (K_SC // IMG_B + i, 0, 0, 0)),
        pl.BlockSpec((IMG_B, 1, ROWS, COLS),
                     lambda i: (K_SC // IMG_B + i, 0, 0, 0)),
    ],
    out_specs=pl.BlockSpec(memory_space=pltpu.SMEM),
    out_shape=jax.ShapeDtypeStruct((4,), jnp.float32),
)


def kernel(inputs, targets):
    parts_tc = _flood_mse_tc(inputs, targets)
    parts_sc = _flood_mse_sc(inputs, targets)
    sums = jnp.sum(parts_sc, axis=(0, 2)) + parts_tc
    loss_label = sums[0] / sums[1]
    loss_pred = sums[2] / sums[3]
    return (loss_label + loss_pred, loss_label, loss_pred)


# speedup vs baseline: 1.0202x; 1.0076x over previous
import functools

import jax
import jax.numpy as jnp
from jax import lax
from jax.experimental import pallas as pl
from jax.experimental.pallas import tpu as pltpu
from jax.experimental.pallas import tpu_sc as plsc

NIMG = 16
ROWS = 512
COLS = 512
K_SC = 4
K_TC = NIMG - K_SC
NC = 2
NS = 16
L = 16
NW = NC * NS
ROWS_W = K_SC * ROWS // NW
CHUNKR = 16
NCHUNK = ROWS_W // CHUNKR
BANKS = 4
STEP = BANKS * L

_mesh = plsc.VectorSubcoreMesh(core_axis_name="c", subcore_axis_name="s")


@functools.partial(
    pl.kernel,
    mesh=_mesh,
    out_type=jax.ShapeDtypeStruct((NW, 4, L), jnp.float32),
    scratch_types=[
        pltpu.VMEM((2, CHUNKR, COLS), jnp.float32),
        pltpu.VMEM((2, CHUNKR, COLS), jnp.float32),
        pltpu.VMEM((4, L), jnp.float32),
        pltpu.SemaphoreType.DMA,
        pltpu.SemaphoreType.DMA,
        pltpu.SemaphoreType.DMA,
        pltpu.SemaphoreType.DMA,
    ],
)
def _flood_mse_sc(a_hbm, b_hbm, out_hbm, a_v, b_v, res_v,
                  sa0, sa1, sb0, sb1):
    wid = lax.axis_index("s") * NC + lax.axis_index("c")
    grow0 = wid * ROWS_W
    sems_a = (sa0, sa1)
    sems_b = (sb0, sb1)

    def dma_pair(c, slot):
        gr = grow0 + c * CHUNKR
        img = gr // ROWS
        r = gr % ROWS
        ca = pltpu.make_async_copy(
            a_hbm.at[img, 0, pl.ds(r, CHUNKR), :], a_v.at[slot],
            sems_a[slot])
        cb = pltpu.make_async_copy(
            b_hbm.at[img, 0, pl.ds(r, CHUNKR), :], b_v.at[slot],
            sems_b[slot])
        return ca, cb

    for c in (0, 1):
        ca, cb = dma_pair(c, c)
        ca.start()
        cb.start()

    zf = jnp.zeros((L,), jnp.float32)
    zi = jnp.zeros((L,), jnp.int32)
    accs = tuple((zf, zi, zf, zi) for _ in range(BANKS))

    for c in range(NCHUNK):
        slot = c % 2
        ca, cb = dma_pair(c, slot)
        ca.wait()
        cb.wait()
        av = a_v.at[slot]
        bv = b_v.at[slot]

        @plsc.parallel_loop(0, CHUNKR * COLS, STEP, unroll=1, carry=accs)
        def accs(i, banks):
            r = jax.lax.shift_right_logical(i, 9)
            col = pl.multiple_of(jax.lax.bitwise_and(i, COLS - 1), STEP)
            out = []
            for j in range(BANKS):
                sl, cl, sp, cp = banks[j]
                a = av[r, pl.ds(col + j * L, L)]
                b = bv[r, pl.ds(col + j * L, L)]
                d = a - b
                sq = d * d
                ml = b > 0.0
                mp = a > 0.0
                sl = sl + jnp.where(ml, sq, 0.0)
                sp = sp + jnp.where(mp, sq, 0.0)
                cl = cl + jnp.where(ml, 1, 0)
                cp = cp + jnp.where(mp, 1, 0)
                out.append((sl, cl, sp, cp))
            return tuple(out)

        if c + 2 < NCHUNK:
            na, nb = dma_pair(c + 2, slot)
            na.start()
            nb.start()

    sl, cl, sp, cp = accs[0]
    for j in range(1, BANKS):
        sl = sl + accs[j][0]
        cl = cl + accs[j][1]
        sp = sp + accs[j][2]
        cp = cp + accs[j][3]
    res_v[0, :] = sl
    res_v[1, :] = cl.astype(jnp.float32)
    res_v[2, :] = sp
    res_v[3, :] = cp.astype(jnp.float32)
    pltpu.sync_copy(res_v, out_hbm.at[wid])


IMG_B = 2


def _flood_mse_tc_body(a_ref, b_ref, o_ref):
    i = pl.program_id(0)
    a = a_ref[...].reshape(IMG_B * ROWS, COLS)
    b = b_ref[...].reshape(IMG_B * ROWS, COLS)
    d = a - b
    sq = d * d
    mlf = (b > 0.0).astype(jnp.float32)
    mpf = (a > 0.0).astype(jnp.float32)
    ones = jnp.ones((1, IMG_B * ROWS), jnp.float32)

    def colsum(x):
        return jax.lax.dot_general(
            ones, x, (((1,), (0,)), ((), ())),
            preferred_element_type=jnp.float32)

    sl = jnp.sum(colsum(sq * mlf))
    cl = jnp.sum(colsum(mlf))
    sp = jnp.sum(colsum(sq * mpf))
    cp = jnp.sum(colsum(mpf))

    @pl.when(i == 0)
    def _():
        o_ref[0] = 0.0
        o_ref[1] = 0.0
        o_ref[2] = 0.0
        o_ref[3] = 0.0

    o_ref[0] += sl
    o_ref[1] += cl
    o_ref[2] += sp
    o_ref[3] += cp


_flood_mse_tc = pl.pallas_call(
    _flood_mse_tc_body,
    grid=(K_TC // IMG_B,),
    in_specs=[
        pl.BlockSpec((IMG_B, 1, ROWS, COLS),
                     lambda i: (K_SC // IMG_B + i, 0, 0, 0)),
        pl.BlockSpec((IMG_B, 1, ROWS, COLS),
                     lambda i: (K_SC // IMG_B + i, 0, 0, 0)),
    ],
    out_specs=pl.BlockSpec(memory_space=pltpu.SMEM),
    out_shape=jax.ShapeDtypeStruct((4,), jnp.float32),
)


def kernel(inputs, targets):
    parts_tc = _flood_mse_tc(inputs, targets)
    parts_sc = _flood_mse_sc(inputs, targets)
    sums = jnp.sum(parts_sc, axis=(0, 2)) + parts_tc
    loss_label = sums[0] / sums[1]
    loss_pred = sums[2] / sums[3]
    return (loss_label + loss_pred, loss_label, loss_pred)


# speedup vs baseline: 1.0229x; 1.0027x over previous
import functools

import jax
import jax.numpy as jnp
from jax import lax
from jax.experimental import pallas as pl
from jax.experimental.pallas import tpu as pltpu
from jax.experimental.pallas import tpu_sc as plsc

NIMG = 16
ROWS = 512
COLS = 512
K_SC = 4
K_TC = NIMG - K_SC
NC = 2
NS = 16
L = 16
NW = NC * NS
ROWS_W = K_SC * ROWS // NW
CHUNKR = 8
NCHUNK = ROWS_W // CHUNKR
BANKS = 4
STEP = BANKS * L

_mesh = plsc.VectorSubcoreMesh(core_axis_name="c", subcore_axis_name="s")


@functools.partial(
    pl.kernel,
    mesh=_mesh,
    out_type=jax.ShapeDtypeStruct((NW, 4, L), jnp.float32),
    scratch_types=[
        pltpu.VMEM((2, CHUNKR, COLS), jnp.float32),
        pltpu.VMEM((2, CHUNKR, COLS), jnp.float32),
        pltpu.VMEM((4, L), jnp.float32),
        pltpu.SemaphoreType.DMA,
        pltpu.SemaphoreType.DMA,
        pltpu.SemaphoreType.DMA,
        pltpu.SemaphoreType.DMA,
    ],
)
def _flood_mse_sc(a_hbm, b_hbm, out_hbm, a_v, b_v, res_v,
                  sa0, sa1, sb0, sb1):
    wid = lax.axis_index("s") * NC + lax.axis_index("c")
    grow0 = wid * ROWS_W
    sems_a = (sa0, sa1)
    sems_b = (sb0, sb1)

    def dma_pair(c, slot):
        gr = grow0 + c * CHUNKR
        img = gr // ROWS
        r = gr % ROWS
        ca = pltpu.make_async_copy(
            a_hbm.at[img, 0, pl.ds(r, CHUNKR), :], a_v.at[slot],
            sems_a[slot])
        cb = pltpu.make_async_copy(
            b_hbm.at[img, 0, pl.ds(r, CHUNKR), :], b_v.at[slot],
            sems_b[slot])
        return ca, cb

    for c in (0, 1):
        ca, cb = dma_pair(c, c)
        ca.start()
        cb.start()

    zf = jnp.zeros((L,), jnp.float32)
    zi = jnp.zeros((L,), jnp.int32)
    accs = tuple((zf, zi, zf, zi) for _ in range(BANKS))

    for c in range(NCHUNK):
        slot = c % 2
        ca, cb = dma_pair(c, slot)
        ca.wait()
        cb.wait()
        av = a_v.at[slot]
        bv = b_v.at[slot]

        @plsc.parallel_loop(0, CHUNKR * COLS, STEP, unroll=1, carry=accs)
        def accs(i, banks):
            r = jax.lax.shift_right_logical(i, 9)
            col = pl.multiple_of(jax.lax.bitwise_and(i, COLS - 1), STEP)
            out = []
            for j in range(BANKS):
                sl, cl, sp, cp = banks[j]
                a = av[r, pl.ds(col + j * L, L)]
                b = bv[r, pl.ds(col + j * L, L)]
                d = a - b
                sq = d * d
                ml = b > 0.0
                mp = a > 0.0
                sl = sl + jnp.where(ml, sq, 0.0)
                sp = sp + jnp.where(mp, sq, 0.0)
                cl = cl + jnp.where(ml, 1, 0)
                cp = cp + jnp.where(mp, 1, 0)
                out.append((sl, cl, sp, cp))
            return tuple(out)

        if c + 2 < NCHUNK:
            na, nb = dma_pair(c + 2, slot)
            na.start()
            nb.start()

    sl, cl, sp, cp = accs[0]
    for j in range(1, BANKS):
        sl = sl + accs[j][0]
        cl = cl + accs[j][1]
        sp = sp + accs[j][2]
        cp = cp + accs[j][3]
    res_v[0, :] = sl
    res_v[1, :] = cl.astype(jnp.float32)
    res_v[2, :] = sp
    res_v[3, :] = cp.astype(jnp.float32)
    pltpu.sync_copy(res_v, out_hbm.at[wid])


IMG_B = 2


def _flood_mse_tc_body(a_ref, b_ref, o_ref):
    i = pl.program_id(0)
    a = a_ref[...].reshape(IMG_B * ROWS, COLS)
    b = b_ref[...].reshape(IMG_B * ROWS, COLS)
    d = a - b
    sq = d * d
    mlf = (b > 0.0).astype(jnp.float32)
    mpf = (a > 0.0).astype(jnp.float32)
    ones = jnp.ones((1, IMG_B * ROWS), jnp.float32)

    def colsum(x):
        return jax.lax.dot_general(
            ones, x, (((1,), (0,)), ((), ())),
            preferred_element_type=jnp.float32)

    sl = jnp.sum(colsum(sq * mlf))
    cl = jnp.sum(colsum(mlf))
    sp = jnp.sum(colsum(sq * mpf))
    cp = jnp.sum(colsum(mpf))

    @pl.when(i == 0)
    def _():
        o_ref[0] = 0.0
        o_ref[1] = 0.0
        o_ref[2] = 0.0
        o_ref[3] = 0.0

    o_ref[0] += sl
    o_ref[1] += cl
    o_ref[2] += sp
    o_ref[3] += cp


_flood_mse_tc = pl.pallas_call(
    _flood_mse_tc_body,
    grid=(K_TC // IMG_B,),
    in_specs=[
        pl.BlockSpec((IMG_B, 1, ROWS, COLS),
                     lambda i: (K_SC // IMG_B + i, 0, 0, 0)),
        pl.BlockSpec((IMG_B, 1, ROWS, COLS),
                     lambda i: (K_SC // IMG_B + i, 0, 0, 0)),
    ],
    out_specs=pl.BlockSpec(memory_space=pltpu.SMEM),
    out_shape=jax.ShapeDtypeStruct((4,), jnp.float32),
)


def kernel(inputs, targets):
    parts_tc = _flood_mse_tc(inputs, targets)
    parts_sc = _flood_mse_sc(inputs, targets)
    sums = jnp.sum(parts_sc, axis=(0, 2)) + parts_tc
    loss_label = sums[0] / sums[1]
    loss_pred = sums[2] / sums[3]
    return (loss_label + loss_pred, loss_label, loss_pred)
